# Initial kernel scaffold; baseline (speedup 1.0000x reference)
import jax
import jax.numpy as jnp
from jax.experimental import pallas as pl


def kernel(x, edge_index, W1, b1, W2, b2, W3, b3):
    raise NotImplementedError("write your pallas kernel here")


# speedup vs baseline: 7.7643x; 7.7643x over previous
import functools

import jax
import jax.numpy as jnp
from jax import lax
from jax.experimental import pallas as pl
from jax.experimental.pallas import tpu as pltpu
from jax.experimental.pallas import tpu_sc as plsc

N = 10000
NPAD = 10112
RPT = NPAD // 16
E = 320000
EPAD = 327680
CHUNK = 128
CPT_A = EPAD // 16 // CHUNK
CPT_E = EPAD // 32 // CHUNK
G = 16

_mesh = plsc.VectorSubcoreMesh(core_axis_name="c", subcore_axis_name="s")


@functools.partial(
    pl.kernel,
    out_type=jax.ShapeDtypeStruct((2 * NPAD, 128), jnp.float32),
    mesh=_mesh,
    scratch_types=[
        pltpu.VMEM((CHUNK, 128), jnp.float32),
        pltpu.VMEM((CPT_E, CHUNK), jnp.int32),
        pltpu.VMEM_SHARED((NPAD, 128), jnp.float32),
    ],
)
def _deg_sc(dst_hbm, ones_hbm, zeros_hbm, out_hbm, ones_v, didx_v, acc):
    cid = lax.axis_index("c")
    sid = lax.axis_index("s")
    pltpu.sync_copy(ones_hbm, ones_v)
    pltpu.sync_copy(dst_hbm.at[cid, sid], didx_v)
    pltpu.sync_copy(zeros_hbm, acc.at[pl.ds(sid * RPT, RPT)])
    plsc.subcore_barrier()

    @pl.loop(0, CPT_E)
    def _(i):
        pltpu.sync_copy(ones_v, acc.at[didx_v.at[i]], add=True)

    plsc.subcore_barrier()
    pltpu.sync_copy(
        acc.at[pl.ds(sid * RPT, RPT)],
        out_hbm.at[pl.ds(cid * NPAD + sid * RPT, RPT)],
    )


@functools.partial(
    pl.kernel,
    out_type=jax.ShapeDtypeStruct((2 * NPAD, 128), jnp.float32),
    mesh=_mesh,
    scratch_types=[
        pltpu.VMEM((G, CHUNK), jnp.int32),
        pltpu.VMEM((G, CHUNK), jnp.int32),
        pltpu.VMEM((CHUNK, 128), jnp.float32),
        pltpu.VMEM_SHARED((NPAD, 128), jnp.float32),
        pltpu.SemaphoreType.DMA,
    ],
)
def _agg_es(h_hbm, src_hbm, dst_hbm, zeros_hbm, out_hbm,
            sidx_v, didx_v, rows_v, acc, gsem):
    cid = lax.axis_index("c")
    sid = lax.axis_index("s")

    @pl.when(cid == 0)
    def _():
        pltpu.sync_copy(h_hbm.at[pl.ds(sid * RPT, RPT)],
                        acc.at[pl.ds(sid * RPT, RPT)])

    @pl.when(cid == 1)
    def _():
        pltpu.sync_copy(zeros_hbm, acc.at[pl.ds(sid * RPT, RPT)])

    plsc.subcore_barrier()

    @pl.loop(0, CPT_E // G)
    def _(g):
        pltpu.sync_copy(src_hbm.at[cid, sid, pl.ds(g * G, G)], sidx_v)
        pltpu.sync_copy(dst_hbm.at[cid, sid, pl.ds(g * G, G)], didx_v)
        for j in range(G):
            pltpu.async_copy(h_hbm.at[sidx_v.at[j]], rows_v, gsem).wait()
            pltpu.sync_copy(rows_v, acc.at[didx_v.at[j]], add=True)

    plsc.subcore_barrier()
    pltpu.sync_copy(
        acc.at[pl.ds(sid * RPT, RPT)],
        out_hbm.at[pl.ds(cid * NPAD + sid * RPT, RPT)],
    )


@functools.partial(
    pl.kernel,
    out_type=jax.ShapeDtypeStruct((2 * NPAD, 128), jnp.float32),
    mesh=_mesh,
    scratch_types=[
        pltpu.VMEM((G, CHUNK), jnp.int32),
        pltpu.VMEM((G, CHUNK), jnp.int32),
        pltpu.VMEM((CHUNK, 128), jnp.float32),
        pltpu.VMEM_SHARED((NPAD, 128), jnp.float32),
        pltpu.SemaphoreType.DMA,
    ],
)
def _agg_cs(h_hbm, src_hbm, dst_hbm, out_hbm, sidx_v, didx_v, rows_v, acc, gsem):
    cid = lax.axis_index("c")
    sid = lax.axis_index("s")
    pltpu.sync_copy(
        h_hbm.at[pl.ds(cid * NPAD + sid * RPT, RPT)],
        acc.at[pl.ds(sid * RPT, RPT)],
    )
    plsc.subcore_barrier()

    @pl.loop(0, CPT_A // G)
    def _(g):
        pltpu.sync_copy(src_hbm.at[cid, sid, pl.ds(g * G, G)], sidx_v)
        pltpu.sync_copy(dst_hbm.at[sid, pl.ds(g * G, G)], didx_v)
        for j in range(G):
            pltpu.async_copy(h_hbm.at[sidx_v.at[j]], rows_v, gsem).wait()
            pltpu.sync_copy(rows_v, acc.at[didx_v.at[j]], add=True)

    plsc.subcore_barrier()
    pltpu.sync_copy(
        acc.at[pl.ds(sid * RPT, RPT)],
        out_hbm.at[pl.ds(cid * NPAD + sid * RPT, RPT)],
    )


def _tc1(degp_ref, x_ref, dinv_ref, xs_ref):
    d = degp_ref[...]
    deg = d[:NPAD, 0:1] + d[NPAD:, 0:1] + 1.0
    dinv = lax.rsqrt(deg)
    dinv_ref[...] = dinv
    xs_ref[...] = x_ref[...] * dinv


def _tc2(agg_ref, dinv_ref, w1_ref, b1_ref, out_ref):
    dinv = dinv_ref[...]
    a = (agg_ref[:NPAD, :] + agg_ref[NPAD:, :]) * dinv
    t = jnp.dot(a, w1_ref[...], preferred_element_type=jnp.float32) + b1_ref[...]
    h = jnp.maximum(t, 0.0) * dinv
    out_ref[:NPAD, :] = h[:, :128]
    out_ref[NPAD:, :] = h[:, 128:]


def _tc3(agg_ref, dinv_ref, w2_ref, b2_ref, w3_ref, out_ref):
    dinv = dinv_ref[...]
    a0 = agg_ref[:NPAD, :] * dinv
    a1 = agg_ref[NPAD:, :] * dinv
    t = (jnp.dot(a0, w2_ref[:128, :], preferred_element_type=jnp.float32)
         + jnp.dot(a1, w2_ref[128:, :], preferred_element_type=jnp.float32)
         + b2_ref[...])
    h = jnp.maximum(t, 0.0)
    u = jnp.dot(h, w3_ref[...], preferred_element_type=jnp.float32) * dinv
    out_ref[...] = jnp.concatenate([u, jnp.zeros_like(u)], axis=1)


def _tc4(agg_ref, dinv_ref, b3_ref, out_ref):
    dinv = dinv_ref[...]
    a = (agg_ref[:NPAD, :64] + agg_ref[NPAD:, :64])
    t = a * dinv + b3_ref[...]
    m = jnp.max(t, axis=1, keepdims=True)
    z = t - m
    lse = jnp.log(jnp.sum(jnp.exp(z), axis=1, keepdims=True))
    out_ref[...] = (z - lse)[:N]


def _tc_call(body, out_shapes, *args):
    return pl.pallas_call(body, out_shape=out_shapes)(*args)


def kernel(x, edge_index, W1, b1, W2, b2, W3, b3):
    f32 = jnp.float32
    src = edge_index[0].astype(jnp.int32)
    dst = edge_index[1].astype(jnp.int32)
    padv = jnp.full((EPAD - E,), N, jnp.int32)
    srcp = jnp.concatenate([src, padv])
    dstp = jnp.concatenate([dst, padv])
    src_es = srcp.reshape(2, 16, CPT_E, CHUNK)
    dst_es = dstp.reshape(2, 16, CPT_E, CHUNK)
    src_cs = jnp.stack([srcp, srcp + NPAD]).reshape(2, 16, CPT_A, CHUNK)
    dst_cs = dstp.reshape(16, CPT_A, CHUNK)
    x_pad = jnp.zeros((NPAD, 128), f32).at[:N, :].set(x)
    ones128 = jnp.ones((CHUNK, 128), f32)
    zeros128 = jnp.zeros((RPT, 128), f32)

    degp = _deg_sc(dst_es, ones128, zeros128)
    dinv, xs = _tc_call(
        _tc1,
        (jax.ShapeDtypeStruct((NPAD, 1), f32),
         jax.ShapeDtypeStruct((NPAD, 128), f32)),
        degp, x_pad)

    a1 = _agg_es(xs, src_es, dst_es, zeros128)
    h1 = _tc_call(_tc2, jax.ShapeDtypeStruct((2 * NPAD, 128), f32),
                  a1, dinv, W1, b1.reshape(1, -1))

    a2 = _agg_cs(h1, src_cs, dst_cs)
    u = _tc_call(_tc3, jax.ShapeDtypeStruct((NPAD, 128), f32),
                 a2, dinv, W2, b2.reshape(1, -1), W3)

    a3 = _agg_es(u, src_es, dst_es, zeros128)
    out = _tc_call(_tc4, jax.ShapeDtypeStruct((N, 64), f32),
                   a3, dinv, b3.reshape(1, -1))
    return out


# speedup vs baseline: 23.0672x; 2.9709x over previous
import functools

import jax
import jax.numpy as jnp
from jax import lax
from jax.experimental import pallas as pl
from jax.experimental.pallas import tpu as pltpu
from jax.experimental.pallas import tpu_sc as plsc

N = 10000
NPAD = 10112
RPT = NPAD // 16
E = 320000
EPAD = 327680
CHUNK = 128
CPT_A = EPAD // 16 // CHUNK
CPT_E = EPAD // 32 // CHUNK
G = 16

_mesh = plsc.VectorSubcoreMesh(core_axis_name="c", subcore_axis_name="s")


@functools.partial(
    pl.kernel,
    out_type=jax.ShapeDtypeStruct((2 * NPAD, 128), jnp.float32),
    mesh=_mesh,
    scratch_types=[
        pltpu.VMEM((CHUNK, 128), jnp.float32),
        pltpu.VMEM((CPT_E, CHUNK), jnp.int32),
        pltpu.VMEM_SHARED((NPAD, 128), jnp.float32),
    ],
)
def _deg_sc(dst_hbm, ones_hbm, zeros_hbm, out_hbm, ones_v, didx_v, acc):
    cid = lax.axis_index("c")
    sid = lax.axis_index("s")
    pltpu.sync_copy(ones_hbm, ones_v)
    pltpu.sync_copy(dst_hbm.at[cid, sid], didx_v)
    pltpu.sync_copy(zeros_hbm, acc.at[pl.ds(sid * RPT, RPT)])
    plsc.subcore_barrier()

    @pl.loop(0, CPT_E)
    def _(i):
        pltpu.sync_copy(ones_v, acc.at[didx_v.at[i]], add=True)

    plsc.subcore_barrier()
    pltpu.sync_copy(
        acc.at[pl.ds(sid * RPT, RPT)],
        out_hbm.at[pl.ds(cid * NPAD + sid * RPT, RPT)],
    )


@functools.partial(
    pl.kernel,
    out_type=jax.ShapeDtypeStruct((2 * NPAD, 128), jnp.float32),
    mesh=_mesh,
    scratch_types=[
        pltpu.VMEM((G, CHUNK), jnp.int32),
        pltpu.VMEM((G, CHUNK), jnp.int32),
        pltpu.VMEM((CHUNK, 128), jnp.float32),
        pltpu.VMEM((CHUNK, 128), jnp.float32),
        pltpu.VMEM_SHARED((NPAD, 128), jnp.float32),
        pltpu.SemaphoreType.DMA,
        pltpu.SemaphoreType.DMA,
    ],
)
def _agg_es(h_hbm, src_hbm, dst_hbm, zeros_hbm, out_hbm,
            sidx_v, didx_v, rows_a, rows_b, acc, sem_a, sem_b):
    cid = lax.axis_index("c")
    sid = lax.axis_index("s")

    @pl.when(cid == 0)
    def _():
        pltpu.sync_copy(h_hbm.at[pl.ds(sid * RPT, RPT)],
                        acc.at[pl.ds(sid * RPT, RPT)])

    @pl.when(cid == 1)
    def _():
        pltpu.sync_copy(zeros_hbm, acc.at[pl.ds(sid * RPT, RPT)])

    plsc.subcore_barrier()

    @pl.loop(0, CPT_E // G)
    def _(g):
        pltpu.sync_copy(src_hbm.at[cid, sid, pl.ds(g * G, G)], sidx_v)
        pltpu.sync_copy(dst_hbm.at[cid, sid, pl.ds(g * G, G)], didx_v)
        bufs = (rows_a, rows_b)
        sems = (sem_a, sem_b)
        pend = pltpu.async_copy(h_hbm.at[sidx_v.at[0]], bufs[0], sems[0])
        for j in range(G):
            cur = j % 2
            nxt = 1 - cur
            cur_desc = pend
            if j + 1 < G:
                pend = pltpu.async_copy(
                    h_hbm.at[sidx_v.at[j + 1]], bufs[nxt], sems[nxt])
            cur_desc.wait()
            pltpu.sync_copy(bufs[cur], acc.at[didx_v.at[j]], add=True)

    plsc.subcore_barrier()
    pltpu.sync_copy(
        acc.at[pl.ds(sid * RPT, RPT)],
        out_hbm.at[pl.ds(cid * NPAD + sid * RPT, RPT)],
    )


@functools.partial(
    pl.kernel,
    out_type=jax.ShapeDtypeStruct((2 * NPAD, 128), jnp.float32),
    mesh=_mesh,
    scratch_types=[
        pltpu.VMEM((G, CHUNK), jnp.int32),
        pltpu.VMEM((G, CHUNK), jnp.int32),
        pltpu.VMEM((CHUNK, 128), jnp.float32),
        pltpu.VMEM((CHUNK, 128), jnp.float32),
        pltpu.VMEM_SHARED((NPAD, 128), jnp.float32),
        pltpu.SemaphoreType.DMA,
        pltpu.SemaphoreType.DMA,
    ],
)
def _agg_cs(h_hbm, src_hbm, dst_hbm, out_hbm,
            sidx_v, didx_v, rows_a, rows_b, acc, sem_a, sem_b):
    cid = lax.axis_index("c")
    sid = lax.axis_index("s")
    pltpu.sync_copy(
        h_hbm.at[pl.ds(cid * NPAD + sid * RPT, RPT)],
        acc.at[pl.ds(sid * RPT, RPT)],
    )
    plsc.subcore_barrier()

    @pl.loop(0, CPT_A // G)
    def _(g):
        pltpu.sync_copy(src_hbm.at[cid, sid, pl.ds(g * G, G)], sidx_v)
        pltpu.sync_copy(dst_hbm.at[sid, pl.ds(g * G, G)], didx_v)
        bufs = (rows_a, rows_b)
        sems = (sem_a, sem_b)
        pend = pltpu.async_copy(h_hbm.at[sidx_v.at[0]], bufs[0], sems[0])
        for j in range(G):
            cur = j % 2
            nxt = 1 - cur
            cur_desc = pend
            if j + 1 < G:
                pend = pltpu.async_copy(
                    h_hbm.at[sidx_v.at[j + 1]], bufs[nxt], sems[nxt])
            cur_desc.wait()
            pltpu.sync_copy(bufs[cur], acc.at[didx_v.at[j]], add=True)

    plsc.subcore_barrier()
    pltpu.sync_copy(
        acc.at[pl.ds(sid * RPT, RPT)],
        out_hbm.at[pl.ds(cid * NPAD + sid * RPT, RPT)],
    )


def _tc1(degp_ref, x_ref, dinv_ref, xs_ref):
    d = degp_ref[...]
    deg = d[:NPAD, 0:1] + d[NPAD:, 0:1] + 1.0
    dinv = lax.rsqrt(deg)
    dinv_ref[...] = dinv
    xs_ref[...] = x_ref[...] * dinv


def _tc2(agg_ref, dinv_ref, w1_ref, b1_ref, out_ref):
    dinv = dinv_ref[...]
    a = (agg_ref[:NPAD, :] + agg_ref[NPAD:, :]) * dinv
    t = jnp.dot(a, w1_ref[...], preferred_element_type=jnp.float32) + b1_ref[...]
    h = jnp.maximum(t, 0.0) * dinv
    out_ref[:NPAD, :] = h[:, :128]
    out_ref[NPAD:, :] = h[:, 128:]


def _tc3(agg_ref, dinv_ref, w2_ref, b2_ref, w3_ref, out_ref):
    dinv = dinv_ref[...]
    a0 = agg_ref[:NPAD, :] * dinv
    a1 = agg_ref[NPAD:, :] * dinv
    t = (jnp.dot(a0, w2_ref[:128, :], preferred_element_type=jnp.float32)
         + jnp.dot(a1, w2_ref[128:, :], preferred_element_type=jnp.float32)
         + b2_ref[...])
    h = jnp.maximum(t, 0.0)
    u = jnp.dot(h, w3_ref[...], preferred_element_type=jnp.float32) * dinv
    out_ref[...] = jnp.concatenate([u, jnp.zeros_like(u)], axis=1)


def _tc4(agg_ref, dinv_ref, b3_ref, out_ref):
    dinv = dinv_ref[...]
    a = (agg_ref[:NPAD, :64] + agg_ref[NPAD:, :64])
    t = a * dinv + b3_ref[...]
    m = jnp.max(t, axis=1, keepdims=True)
    z = t - m
    lse = jnp.log(jnp.sum(jnp.exp(z), axis=1, keepdims=True))
    out_ref[...] = (z - lse)[:N]


def _tc_call(body, out_shapes, *args):
    return pl.pallas_call(body, out_shape=out_shapes)(*args)


def kernel(x, edge_index, W1, b1, W2, b2, W3, b3):
    f32 = jnp.float32
    src = edge_index[0].astype(jnp.int32)
    dst = edge_index[1].astype(jnp.int32)
    padv = N + jnp.arange(EPAD - E, dtype=jnp.int32) % (NPAD - N)
    srcp = jnp.concatenate([src, padv])
    dstp = jnp.concatenate([dst, padv])
    src_es = srcp.reshape(2, 16, CPT_E, CHUNK)
    dst_es = dstp.reshape(2, 16, CPT_E, CHUNK)
    src_cs = jnp.stack([srcp, srcp + NPAD]).reshape(2, 16, CPT_A, CHUNK)
    dst_cs = dstp.reshape(16, CPT_A, CHUNK)
    x_pad = jnp.zeros((NPAD, 128), f32).at[:N, :].set(x)
    ones128 = jnp.ones((CHUNK, 128), f32)
    zeros128 = jnp.zeros((RPT, 128), f32)

    degp = _deg_sc(dst_es, ones128, zeros128)
    dinv, xs = _tc_call(
        _tc1,
        (jax.ShapeDtypeStruct((NPAD, 1), f32),
         jax.ShapeDtypeStruct((NPAD, 128), f32)),
        degp, x_pad)

    a1 = _agg_es(xs, src_es, dst_es, zeros128)
    h1 = _tc_call(_tc2, jax.ShapeDtypeStruct((2 * NPAD, 128), f32),
                  a1, dinv, W1, b1.reshape(1, -1))

    a2 = _agg_cs(h1, src_cs, dst_cs)
    u = _tc_call(_tc3, jax.ShapeDtypeStruct((NPAD, 128), f32),
                 a2, dinv, W2, b2.reshape(1, -1), W3)

    a3 = _agg_es(u, src_es, dst_es, zeros128)
    out = _tc_call(_tc4, jax.ShapeDtypeStruct((N, 64), f32),
                   a3, dinv, b3.reshape(1, -1))
    return out


# speedup vs baseline: 23.1069x; 1.0017x over previous
import functools

import jax
import jax.numpy as jnp
from jax import lax
from jax.experimental import pallas as pl
from jax.experimental.pallas import tpu as pltpu
from jax.experimental.pallas import tpu_sc as plsc

N = 10000
NPAD = 10112
RPT = NPAD // 16
E = 320000
EPAD = 327680
CHUNK = 128
CPT_A = EPAD // 16 // CHUNK
CPT_E = EPAD // 32 // CHUNK
G = 16

_mesh = plsc.VectorSubcoreMesh(core_axis_name="c", subcore_axis_name="s")


@functools.partial(
    pl.kernel,
    out_type=jax.ShapeDtypeStruct((2 * NPAD, 128), jnp.float32),
    mesh=_mesh,
    scratch_types=[
        pltpu.VMEM((CHUNK, 128), jnp.float32),
        pltpu.VMEM((CPT_E, CHUNK), jnp.int32),
        pltpu.VMEM_SHARED((NPAD, 128), jnp.float32),
        pltpu.SemaphoreType.DMA,
    ],
)
def _deg_sc(dst_hbm, ones_hbm, zeros_hbm, out_hbm, ones_v, didx_v, acc, ssem):
    cid = lax.axis_index("c")
    sid = lax.axis_index("s")
    pltpu.sync_copy(ones_hbm, ones_v)
    pltpu.sync_copy(dst_hbm.at[cid, sid], didx_v)
    pltpu.sync_copy(zeros_hbm, acc.at[pl.ds(sid * RPT, RPT)])
    plsc.subcore_barrier()

    @pl.loop(0, CPT_E // G)
    def _(g):
        descs = []
        for j in range(G):
            descs.append(pltpu.async_copy(
                ones_v, acc.at[didx_v.at[g * G + j]], ssem, add=True))
        for dsc in descs:
            dsc.wait()

    plsc.subcore_barrier()
    pltpu.sync_copy(
        acc.at[pl.ds(sid * RPT, RPT)],
        out_hbm.at[pl.ds(cid * NPAD + sid * RPT, RPT)],
    )


@functools.partial(
    pl.kernel,
    out_type=jax.ShapeDtypeStruct((2 * NPAD, 128), jnp.float32),
    mesh=_mesh,
    scratch_types=[
        pltpu.VMEM((G, CHUNK), jnp.int32),
        pltpu.VMEM((G, CHUNK), jnp.int32),
        pltpu.VMEM((CHUNK, 128), jnp.float32),
        pltpu.VMEM((CHUNK, 128), jnp.float32),
        pltpu.VMEM_SHARED((NPAD, 128), jnp.float32),
        pltpu.SemaphoreType.DMA,
        pltpu.SemaphoreType.DMA,
        pltpu.SemaphoreType.DMA,
        pltpu.SemaphoreType.DMA,
    ],
)
def _agg_es(h_hbm, src_hbm, dst_hbm, zeros_hbm, out_hbm,
            sidx_v, didx_v, rows_a, rows_b, acc, sem_a, sem_b, sem_c, sem_d):
    cid = lax.axis_index("c")
    sid = lax.axis_index("s")

    @pl.when(cid == 0)
    def _():
        pltpu.sync_copy(h_hbm.at[pl.ds(sid * RPT, RPT)],
                        acc.at[pl.ds(sid * RPT, RPT)])

    @pl.when(cid == 1)
    def _():
        pltpu.sync_copy(zeros_hbm, acc.at[pl.ds(sid * RPT, RPT)])

    plsc.subcore_barrier()

    @pl.loop(0, CPT_E // G)
    def _(g):
        pltpu.sync_copy(src_hbm.at[cid, sid, pl.ds(g * G, G)], sidx_v)
        pltpu.sync_copy(dst_hbm.at[cid, sid, pl.ds(g * G, G)], didx_v)
        bufs = (rows_a, rows_b)
        gsems = (sem_a, sem_b)
        ssems = (sem_c, sem_d)
        pend_g = [pltpu.async_copy(h_hbm.at[sidx_v.at[0]], bufs[0], gsems[0]),
                  None]
        pend_s = [None, None]
        for j in range(G):
            cur = j % 2
            nxt = 1 - cur
            if j + 1 < G:
                if pend_s[nxt] is not None:
                    pend_s[nxt].wait()
                    pend_s[nxt] = None
                pend_g[nxt] = pltpu.async_copy(
                    h_hbm.at[sidx_v.at[j + 1]], bufs[nxt], gsems[nxt])
            pend_g[cur].wait()
            pend_s[cur] = pltpu.async_copy(
                bufs[cur], acc.at[didx_v.at[j]], ssems[cur], add=True)
        for dsc in pend_s:
            if dsc is not None:
                dsc.wait()

    plsc.subcore_barrier()
    pltpu.sync_copy(
        acc.at[pl.ds(sid * RPT, RPT)],
        out_hbm.at[pl.ds(cid * NPAD + sid * RPT, RPT)],
    )


@functools.partial(
    pl.kernel,
    out_type=jax.ShapeDtypeStruct((2 * NPAD, 128), jnp.float32),
    mesh=_mesh,
    scratch_types=[
        pltpu.VMEM((G, CHUNK), jnp.int32),
        pltpu.VMEM((G, CHUNK), jnp.int32),
        pltpu.VMEM((CHUNK, 128), jnp.float32),
        pltpu.VMEM((CHUNK, 128), jnp.float32),
        pltpu.VMEM_SHARED((NPAD, 128), jnp.float32),
        pltpu.SemaphoreType.DMA,
        pltpu.SemaphoreType.DMA,
        pltpu.SemaphoreType.DMA,
        pltpu.SemaphoreType.DMA,
    ],
)
def _agg_cs(h_hbm, src_hbm, dst_hbm, out_hbm,
            sidx_v, didx_v, rows_a, rows_b, acc, sem_a, sem_b, sem_c, sem_d):
    cid = lax.axis_index("c")
    sid = lax.axis_index("s")
    pltpu.sync_copy(
        h_hbm.at[pl.ds(cid * NPAD + sid * RPT, RPT)],
        acc.at[pl.ds(sid * RPT, RPT)],
    )
    plsc.subcore_barrier()

    @pl.loop(0, CPT_A // G)
    def _(g):
        pltpu.sync_copy(src_hbm.at[cid, sid, pl.ds(g * G, G)], sidx_v)
        pltpu.sync_copy(dst_hbm.at[sid, pl.ds(g * G, G)], didx_v)
        bufs = (rows_a, rows_b)
        gsems = (sem_a, sem_b)
        ssems = (sem_c, sem_d)
        pend_g = [pltpu.async_copy(h_hbm.at[sidx_v.at[0]], bufs[0], gsems[0]),
                  None]
        pend_s = [None, None]
        for j in range(G):
            cur = j % 2
            nxt = 1 - cur
            if j + 1 < G:
                if pend_s[nxt] is not None:
                    pend_s[nxt].wait()
                    pend_s[nxt] = None
                pend_g[nxt] = pltpu.async_copy(
                    h_hbm.at[sidx_v.at[j + 1]], bufs[nxt], gsems[nxt])
            pend_g[cur].wait()
            pend_s[cur] = pltpu.async_copy(
                bufs[cur], acc.at[didx_v.at[j]], ssems[cur], add=True)
        for dsc in pend_s:
            if dsc is not None:
                dsc.wait()

    plsc.subcore_barrier()
    pltpu.sync_copy(
        acc.at[pl.ds(sid * RPT, RPT)],
        out_hbm.at[pl.ds(cid * NPAD + sid * RPT, RPT)],
    )


def _tc1(degp_ref, x_ref, dinv_ref, xs_ref):
    d = degp_ref[...]
    deg = d[:NPAD, 0:1] + d[NPAD:, 0:1] + 1.0
    dinv = lax.rsqrt(deg)
    dinv_ref[...] = dinv
    xs_ref[...] = x_ref[...] * dinv


def _tc2(agg_ref, dinv_ref, w1_ref, b1_ref, out_ref):
    dinv = dinv_ref[...]
    a = (agg_ref[:NPAD, :] + agg_ref[NPAD:, :]) * dinv
    t = jnp.dot(a, w1_ref[...], preferred_element_type=jnp.float32) + b1_ref[...]
    h = jnp.maximum(t, 0.0) * dinv
    out_ref[:NPAD, :] = h[:, :128]
    out_ref[NPAD:, :] = h[:, 128:]


def _tc3(agg_ref, dinv_ref, w2_ref, b2_ref, w3_ref, out_ref):
    dinv = dinv_ref[...]
    a0 = agg_ref[:NPAD, :] * dinv
    a1 = agg_ref[NPAD:, :] * dinv
    t = (jnp.dot(a0, w2_ref[:128, :], preferred_element_type=jnp.float32)
         + jnp.dot(a1, w2_ref[128:, :], preferred_element_type=jnp.float32)
         + b2_ref[...])
    h = jnp.maximum(t, 0.0)
    u = jnp.dot(h, w3_ref[...], preferred_element_type=jnp.float32) * dinv
    out_ref[...] = jnp.concatenate([u, jnp.zeros_like(u)], axis=1)


def _tc4(agg_ref, dinv_ref, b3_ref, out_ref):
    dinv = dinv_ref[...]
    a = (agg_ref[:NPAD, :64] + agg_ref[NPAD:, :64])
    t = a * dinv + b3_ref[...]
    m = jnp.max(t, axis=1, keepdims=True)
    z = t - m
    lse = jnp.log(jnp.sum(jnp.exp(z), axis=1, keepdims=True))
    out_ref[...] = (z - lse)[:N]


def _tc_call(body, out_shapes, *args):
    return pl.pallas_call(body, out_shape=out_shapes)(*args)


def kernel(x, edge_index, W1, b1, W2, b2, W3, b3):
    f32 = jnp.float32
    src = edge_index[0].astype(jnp.int32)
    dst = edge_index[1].astype(jnp.int32)
    padv = N + jnp.arange(EPAD - E, dtype=jnp.int32) % (NPAD - N)
    srcp = jnp.concatenate([src, padv])
    dstp = jnp.concatenate([dst, padv])
    src_es = srcp.reshape(2, 16, CPT_E, CHUNK)
    dst_es = dstp.reshape(2, 16, CPT_E, CHUNK)
    src_cs = jnp.stack([srcp, srcp + NPAD]).reshape(2, 16, CPT_A, CHUNK)
    dst_cs = dstp.reshape(16, CPT_A, CHUNK)
    x_pad = jnp.zeros((NPAD, 128), f32).at[:N, :].set(x)
    ones128 = jnp.ones((CHUNK, 128), f32)
    zeros128 = jnp.zeros((RPT, 128), f32)

    degp = _deg_sc(dst_es, ones128, zeros128)
    dinv, xs = _tc_call(
        _tc1,
        (jax.ShapeDtypeStruct((NPAD, 1), f32),
         jax.ShapeDtypeStruct((NPAD, 128), f32)),
        degp, x_pad)

    a1 = _agg_es(xs, src_es, dst_es, zeros128)
    h1 = _tc_call(_tc2, jax.ShapeDtypeStruct((2 * NPAD, 128), f32),
                  a1, dinv, W1, b1.reshape(1, -1))

    a2 = _agg_cs(h1, src_cs, dst_cs)
    u = _tc_call(_tc3, jax.ShapeDtypeStruct((NPAD, 128), f32),
                 a2, dinv, W2, b2.reshape(1, -1), W3)

    a3 = _agg_es(u, src_es, dst_es, zeros128)
    out = _tc_call(_tc4, jax.ShapeDtypeStruct((N, 64), f32),
                   a3, dinv, b3.reshape(1, -1))
    return out
